# baseline (speedup 1.0000x reference)
import functools

import jax
import jax.numpy as jnp
from jax import lax
from jax.experimental import pallas as pl
from jax.experimental.pallas import tpu as pltpu
from jax.experimental.pallas import tpu_sc as plsc

NN = 10000
NE = 160000
DF = 128
NW = 32
CH = NE // NW
FULL = CH // 16
TAIL = CH - FULL * 16
CHP = (FULL + 1) * 16
CB = 256
NCHUNK = CH // CB
LASTROWS = CH - NCHUNK * CB

_P = jax.lax.Precision.HIGHEST


def _k1_body(v_ref, wf_ref, wa_ref, b_ref, p_ref, q_ref, a3_ref):
    h = lax.dot_general(v_ref[...], wf_ref[...], (((1,), (1,)), ((), ())),
                        precision=_P, preferred_element_type=jnp.float32)
    p_ref[...] = jnp.sum(h * wa_ref[:, 0:128], axis=1) + b_ref[0, 0]
    q_ref[...] = jnp.sum(h * wa_ref[:, 128:256], axis=1)
    a3_ref[...] = wa_ref[0, 256:384]


def _k4_body(v_ref, wf_ref, dp_ref, mv_ref, o_ref):
    mv = mv_ref[:, 0:1]
    scale = jnp.exp(mv - jnp.max(mv))
    d = jnp.sum(dp_ref[...] * scale, axis=0)
    h = lax.dot_general(v_ref[...], wf_ref[...], (((1,), (1,)), ((), ())),
                        precision=_P, preferred_element_type=jnp.float32)
    o_ref[...] = h * (d / (d + 1e-8))[:, None]


_sc_mesh = plsc.VectorSubcoreMesh(core_axis_name="c", subcore_axis_name="s")
_sc_params = pltpu.CompilerParams(needs_layout_passes=False)


@functools.partial(
    pl.kernel, mesh=_sc_mesh, compiler_params=_sc_params,
    out_type=[jax.ShapeDtypeStruct((NW, NN), jnp.float32),
              jax.ShapeDtypeStruct((NW, 16), jnp.float32)],
    scratch_types=[pltpu.VMEM((NN,), jnp.float32),
                   pltpu.VMEM((NN,), jnp.float32),
                   pltpu.VMEM((CHP,), jnp.int32),
                   pltpu.VMEM((CHP,), jnp.int32),
                   pltpu.VMEM((CHP,), jnp.float32),
                   pltpu.VMEM((NN,), jnp.float32),
                   pltpu.VMEM((16,), jnp.float32),
                   pltpu.VMEM((128,), jnp.float32),
                   pltpu.VMEM((CB, DF), jnp.float32),
                   pltpu.VMEM((CB, DF), jnp.float32),
                   pltpu.VMEM((256,), jnp.float32),
                   pltpu.SemaphoreType.DMA,
                   pltpu.SemaphoreType.DMA])
def _k3(p_hbm, q_hbm, s_hbm, r_hbm, e_hbm, a3_hbm, dpart_hbm, mvec_hbm,
        p_v, q_v, s_v, r_v, lo_v, d_v, mx_v, a3_v, eb0, eb1, tr_v,
        sem0, sem1):
    wid = lax.axis_index("s") * 2 + lax.axis_index("c")
    base = wid * CH

    def start(c, buf, sem):
        pltpu.async_copy(e_hbm.at[pl.ds(base + c * CB, CB)], buf, sem)

    def drain(buf, sem):
        pltpu.make_async_copy(e_hbm.at[pl.ds(base, CB)], buf, sem).wait()

    start(0, eb0, sem0)
    start(1, eb1, sem1)

    pltpu.sync_copy(p_hbm, p_v)
    pltpu.sync_copy(q_hbm, q_v)
    pltpu.sync_copy(s_hbm.at[pl.ds(base, CH)], s_v.at[pl.ds(0, CH)])
    pltpu.sync_copy(r_hbm.at[pl.ds(base, CH)], r_v.at[pl.ds(0, CH)])
    pltpu.sync_copy(a3_hbm, a3_v)

    iota = lax.iota(jnp.int32, 16)
    a3r = [a3_v[pl.ds(16 * k, 16)] for k in range(8)]

    def dot16(buf, row0):
        for e in range(16):
            t = buf[row0 + e, pl.ds(0, 16)] * a3r[0]
            for k in range(1, 8):
                t = t + buf[row0 + e, pl.ds(16 * k, 16)] * a3r[k]
            tr_v[pl.ds(e * 16, 16)] = t
        acc = plsc.load_gather(tr_v, [iota * 16])
        for l in range(1, 16):
            acc = acc + plsc.load_gather(tr_v, [iota * 16 + l])
        return acc

    def group_body(buf, c, g, m16):
        les = dot16(buf, g * 16)
        sl = pl.ds(c * CB + g * 16, 16)
        lg = (plsc.load_gather(p_v, [s_v[sl]])
              + plsc.load_gather(q_v, [r_v[sl]])
              + les)
        lg = jnp.where(lg >= 0.0, lg, 0.2 * lg)
        lo_v[sl] = lg
        return jnp.maximum(m16, lg)

    def groups(buf, c, ng, m16):
        return lax.fori_loop(0, ng, lambda g, m: group_body(buf, c, g, m),
                             m16)

    def jbody(j, m16):
        c0 = 2 * j
        drain(eb0, sem0)
        m16 = groups(eb0, c0, CB // 16, m16)

        @pl.when(j < 9)
        def _():
            start(c0 + 2, eb0, sem0)
        drain(eb1, sem1)
        m16 = groups(eb1, c0 + 1, CB // 16, m16)

        @pl.when(j < 8)
        def _():
            start(c0 + 3, eb1, sem1)
        return m16

    m16 = lax.fori_loop(0, 9, jbody, jnp.full((16,), -3e38, jnp.float32))

    pltpu.async_copy(e_hbm.at[pl.ds(base + NCHUNK * CB, LASTROWS)],
                     eb1.at[pl.ds(0, LASTROWS)], sem1)
    drain(eb0, sem0)
    m16 = groups(eb0, 18, CB // 16, m16)
    pltpu.make_async_copy(e_hbm.at[pl.ds(base, LASTROWS)],
                          eb1.at[pl.ds(0, LASTROWS)], sem1).wait()
    m16 = groups(eb1, 19, TAIL * 16 // 16 // 2, m16)
    tmask = iota < TAIL
    sl = pl.ds(FULL * 16, 16)
    les = dot16(eb1, 8 * 16)
    s16 = jnp.where(tmask, s_v[sl], 0)
    r16 = jnp.where(tmask, r_v[sl], 0)
    lg = (plsc.load_gather(p_v, [s16], mask=tmask)
          + plsc.load_gather(q_v, [r16], mask=tmask)
          + les)
    lg = jnp.where(lg >= 0.0, lg, 0.2 * lg)
    lg = jnp.where(tmask, lg, -3e38)
    s_v[sl] = s16
    lo_v[sl] = lg
    m16 = jnp.maximum(m16, lg)

    ms = jnp.full((16,), jnp.max(m16))
    mx_v[...] = ms

    def zero_body(j, c):
        d_v[pl.ds(j * 16, 16)] = jnp.zeros((16,), jnp.float32)
        return c
    lax.fori_loop(0, NN // 16, zero_body, 0)

    def acc_body(i, c):
        sl = pl.ds(i * 16, 16)
        att = jnp.exp(lo_v[sl] - ms)
        plsc.addupdate_scatter(d_v, [s_v[sl]], att)
        return c
    lax.fori_loop(0, FULL + 1, acc_body, 0)

    pltpu.sync_copy(d_v, dpart_hbm.at[wid])
    pltpu.sync_copy(mx_v, mvec_hbm.at[wid])


def kernel(V, E, edges, W_f, W_a, b_a):
    V2 = V[0]
    E2 = E[0]

    p, q, a3 = pl.pallas_call(
        _k1_body,
        grid=(10,),
        in_specs=[pl.BlockSpec((1024, 128), lambda i: (i, 0)),
                  pl.BlockSpec((128, 128), lambda i: (0, 0)),
                  pl.BlockSpec((1, 384), lambda i: (0, 0)),
                  pl.BlockSpec((1, 1), lambda i: (0, 0))],
        out_specs=[pl.BlockSpec((1024,), lambda i: (i,)),
                   pl.BlockSpec((1024,), lambda i: (i,)),
                   pl.BlockSpec((128,), lambda i: (0,))],
        out_shape=[jax.ShapeDtypeStruct((NN,), jnp.float32),
                   jax.ShapeDtypeStruct((NN,), jnp.float32),
                   jax.ShapeDtypeStruct((128,), jnp.float32)],
    )(V2, W_f, W_a, b_a.reshape(1, 1))

    dpart, mvec = _k3(p, q, edges[0, :, 0], edges[0, :, 1], E2, a3)

    h = pl.pallas_call(
        _k4_body,
        out_shape=jax.ShapeDtypeStruct((NN, 128), jnp.float32),
    )(V2, W_f, dpart, mvec)
    return h.reshape(1, NN, DF)


# speedup vs baseline: 1.0484x; 1.0484x over previous
import functools

import jax
import jax.numpy as jnp
from jax import lax
from jax.experimental import pallas as pl
from jax.experimental.pallas import tpu as pltpu
from jax.experimental.pallas import tpu_sc as plsc

NN = 10000
NE = 160000
DF = 128
NW = 32
CH = NE // NW
FULL = CH // 16
TAIL = CH - FULL * 16
CHP = (FULL + 1) * 16
CB = 256
NCHUNK = CH // CB
LASTROWS = CH - NCHUNK * CB

_P = jax.lax.Precision.HIGHEST


def _k1_body(v_ref, wf_ref, wa_ref, b_ref, p_ref, q_ref, a3_ref):
    h = lax.dot_general(v_ref[...], wf_ref[...], (((1,), (1,)), ((), ())),
                        precision=_P, preferred_element_type=jnp.float32)
    p_ref[...] = jnp.sum(h * wa_ref[:, 0:128], axis=1) + b_ref[0, 0]
    q_ref[...] = jnp.sum(h * wa_ref[:, 128:256], axis=1)
    a3_ref[...] = wa_ref[0, 256:384]


def _k4_body(v_ref, wf_ref, dp_ref, mv_ref, o_ref):
    mv = mv_ref[:, 0:1]
    scale = jnp.exp(mv - jnp.max(mv))
    d = jnp.sum(dp_ref[...] * scale, axis=0)
    h = lax.dot_general(v_ref[...], wf_ref[...], (((1,), (1,)), ((), ())),
                        precision=_P, preferred_element_type=jnp.float32)
    o_ref[...] = h * (d / (d + 1e-8))[:, None]


_sc_mesh = plsc.VectorSubcoreMesh(core_axis_name="c", subcore_axis_name="s")
_sc_params = pltpu.CompilerParams(needs_layout_passes=False)


@functools.partial(
    pl.kernel, mesh=_sc_mesh, compiler_params=_sc_params,
    out_type=[jax.ShapeDtypeStruct((NW, NN), jnp.float32),
              jax.ShapeDtypeStruct((NW, 16), jnp.float32)],
    scratch_types=[pltpu.VMEM((NN,), jnp.float32),
                   pltpu.VMEM((NN,), jnp.float32),
                   pltpu.VMEM((CHP,), jnp.int32),
                   pltpu.VMEM((CHP,), jnp.int32),
                   pltpu.VMEM((CHP,), jnp.float32),
                   pltpu.VMEM((NN,), jnp.float32),
                   pltpu.VMEM((16,), jnp.float32),
                   pltpu.VMEM((128,), jnp.float32),
                   pltpu.VMEM((CB, DF), jnp.float32),
                   pltpu.VMEM((CB, DF), jnp.float32),
                   pltpu.VMEM((256,), jnp.float32),
                   pltpu.SemaphoreType.DMA,
                   pltpu.SemaphoreType.DMA])
def _k3(p_hbm, q_hbm, s_hbm, r_hbm, e_hbm, a3_hbm, dpart_hbm, mvec_hbm,
        p_v, q_v, s_v, r_v, lo_v, d_v, mx_v, a3_v, eb0, eb1, tr_v,
        sem0, sem1):
    wid = lax.axis_index("s") * 2 + lax.axis_index("c")
    base = wid * CH

    def start(c, buf, sem):
        pltpu.async_copy(e_hbm.at[pl.ds(base + c * CB, CB)], buf, sem)

    def drain(buf, sem):
        pltpu.make_async_copy(e_hbm.at[pl.ds(base, CB)], buf, sem).wait()

    start(0, eb0, sem0)
    start(1, eb1, sem1)

    pltpu.sync_copy(p_hbm, p_v)
    pltpu.sync_copy(q_hbm, q_v)
    pltpu.sync_copy(s_hbm.at[pl.ds(base, CH)], s_v.at[pl.ds(0, CH)])
    pltpu.sync_copy(r_hbm.at[pl.ds(base, CH)], r_v.at[pl.ds(0, CH)])
    pltpu.sync_copy(a3_hbm, a3_v)

    iota = lax.iota(jnp.int32, 16)
    a3r = [a3_v[pl.ds(16 * k, 16)] for k in range(8)]

    def dot16(buf, row0):
        for e in range(16):
            prods = [buf[row0 + e, pl.ds(16 * k, 16)] * a3r[k]
                     for k in range(8)]
            while len(prods) > 1:
                prods = [prods[i] + prods[i + 1]
                         for i in range(0, len(prods), 2)]
            tr_v[pl.ds(e * 16, 16)] = prods[0]
        cols = [plsc.load_gather(tr_v, [iota * 16 + l]) for l in range(16)]
        while len(cols) > 1:
            cols = [cols[i] + cols[i + 1] for i in range(0, len(cols), 2)]
        return cols[0]

    def group_body(buf, c, g, m16):
        les = dot16(buf, g * 16)
        sl = pl.ds(c * CB + g * 16, 16)
        lg = (plsc.load_gather(p_v, [s_v[sl]])
              + plsc.load_gather(q_v, [r_v[sl]])
              + les)
        lg = jnp.where(lg >= 0.0, lg, 0.2 * lg)
        lo_v[sl] = lg
        return jnp.maximum(m16, lg)

    def groups(buf, c, ng, m16):
        return lax.fori_loop(0, ng, lambda g, m: group_body(buf, c, g, m),
                             m16)

    def jbody(j, m16):
        c0 = 2 * j
        drain(eb0, sem0)
        m16 = groups(eb0, c0, CB // 16, m16)

        @pl.when(j < 9)
        def _():
            start(c0 + 2, eb0, sem0)
        drain(eb1, sem1)
        m16 = groups(eb1, c0 + 1, CB // 16, m16)

        @pl.when(j < 8)
        def _():
            start(c0 + 3, eb1, sem1)
        return m16

    m16 = lax.fori_loop(0, 9, jbody, jnp.full((16,), -3e38, jnp.float32))

    pltpu.async_copy(e_hbm.at[pl.ds(base + NCHUNK * CB, LASTROWS)],
                     eb1.at[pl.ds(0, LASTROWS)], sem1)
    drain(eb0, sem0)
    m16 = groups(eb0, 18, CB // 16, m16)
    pltpu.make_async_copy(e_hbm.at[pl.ds(base, LASTROWS)],
                          eb1.at[pl.ds(0, LASTROWS)], sem1).wait()
    m16 = groups(eb1, 19, TAIL * 16 // 16 // 2, m16)
    tmask = iota < TAIL
    sl = pl.ds(FULL * 16, 16)
    les = dot16(eb1, 8 * 16)
    s16 = jnp.where(tmask, s_v[sl], 0)
    r16 = jnp.where(tmask, r_v[sl], 0)
    lg = (plsc.load_gather(p_v, [s16], mask=tmask)
          + plsc.load_gather(q_v, [r16], mask=tmask)
          + les)
    lg = jnp.where(lg >= 0.0, lg, 0.2 * lg)
    lg = jnp.where(tmask, lg, -3e38)
    s_v[sl] = s16
    lo_v[sl] = lg
    m16 = jnp.maximum(m16, lg)

    ms = jnp.full((16,), jnp.max(m16))
    mx_v[...] = ms

    def zero_body(j, c):
        d_v[pl.ds(j * 16, 16)] = jnp.zeros((16,), jnp.float32)
        return c
    lax.fori_loop(0, NN // 16, zero_body, 0)

    def acc_body(i, c):
        sl = pl.ds(i * 16, 16)
        att = jnp.exp(lo_v[sl] - ms)
        plsc.addupdate_scatter(d_v, [s_v[sl]], att)
        return c
    lax.fori_loop(0, FULL + 1, acc_body, 0)

    pltpu.sync_copy(d_v, dpart_hbm.at[wid])
    pltpu.sync_copy(mx_v, mvec_hbm.at[wid])


def kernel(V, E, edges, W_f, W_a, b_a):
    V2 = V[0]
    E2 = E[0]

    p, q, a3 = pl.pallas_call(
        _k1_body,
        grid=(10,),
        in_specs=[pl.BlockSpec((1024, 128), lambda i: (i, 0)),
                  pl.BlockSpec((128, 128), lambda i: (0, 0)),
                  pl.BlockSpec((1, 384), lambda i: (0, 0)),
                  pl.BlockSpec((1, 1), lambda i: (0, 0))],
        out_specs=[pl.BlockSpec((1024,), lambda i: (i,)),
                   pl.BlockSpec((1024,), lambda i: (i,)),
                   pl.BlockSpec((128,), lambda i: (0,))],
        out_shape=[jax.ShapeDtypeStruct((NN,), jnp.float32),
                   jax.ShapeDtypeStruct((NN,), jnp.float32),
                   jax.ShapeDtypeStruct((128,), jnp.float32)],
    )(V2, W_f, W_a, b_a.reshape(1, 1))

    dpart, mvec = _k3(p, q, edges[0, :, 0], edges[0, :, 1], E2, a3)

    h = pl.pallas_call(
        _k4_body,
        out_shape=jax.ShapeDtypeStruct((NN, 128), jnp.float32),
    )(V2, W_f, dpart, mvec)
    return h.reshape(1, NN, DF)


# speedup vs baseline: 1.1150x; 1.0635x over previous
import functools

import jax
import jax.numpy as jnp
from jax import lax
from jax.experimental import pallas as pl
from jax.experimental.pallas import tpu as pltpu
from jax.experimental.pallas import tpu_sc as plsc

NN = 10000
NE = 160000
DF = 128
NW = 32
CH = NE // NW
FULL = CH // 16
TAIL = CH - FULL * 16
CHP = (FULL + 1) * 16

_P = jax.lax.Precision.HIGHEST


def _k2_body(e_ref, v_ref, wf_ref, wa_ref, b_ref, le_ref, p_ref, q_ref):
    a3 = wa_ref[:, 256:384]
    le_ref[...] = jnp.sum(e_ref[...] * a3, axis=1) + b_ref[0, 0]
    h = lax.dot_general(v_ref[...], wf_ref[...], (((1,), (1,)), ((), ())),
                        precision=_P, preferred_element_type=jnp.float32)
    p_ref[...] = jnp.sum(h * wa_ref[:, 0:128], axis=1)
    q_ref[...] = jnp.sum(h * wa_ref[:, 128:256], axis=1)


def _k4_body(v_ref, wf_ref, dp_ref, mv_ref, o_ref):
    mv = mv_ref[:, 0:1]
    scale = jnp.exp(mv - jnp.max(mv))
    d = jnp.sum(dp_ref[...] * scale, axis=0)
    h = lax.dot_general(v_ref[...], wf_ref[...], (((1,), (1,)), ((), ())),
                        precision=_P, preferred_element_type=jnp.float32)
    o_ref[...] = h * (d / (d + 1e-8))[:, None]


_sc_mesh = plsc.VectorSubcoreMesh(core_axis_name="c", subcore_axis_name="s")
_sc_params = pltpu.CompilerParams(needs_layout_passes=False)


@functools.partial(
    pl.kernel, mesh=_sc_mesh, compiler_params=_sc_params,
    out_type=[jax.ShapeDtypeStruct((NW, NN), jnp.float32),
              jax.ShapeDtypeStruct((NW, 16), jnp.float32)],
    scratch_types=[pltpu.VMEM((NN,), jnp.float32),
                   pltpu.VMEM((NN,), jnp.float32),
                   pltpu.VMEM((CHP,), jnp.int32),
                   pltpu.VMEM((CHP,), jnp.int32),
                   pltpu.VMEM((CHP,), jnp.float32),
                   pltpu.VMEM((CHP,), jnp.float32),
                   pltpu.VMEM((NN,), jnp.float32),
                   pltpu.VMEM((16,), jnp.float32)])
def _k3(p_hbm, q_hbm, s_hbm, r_hbm, le_hbm, dpart_hbm, mvec_hbm,
        p_v, q_v, s_v, r_v, le_v, lo_v, d_v, mx_v):
    wid = lax.axis_index("s") * 2 + lax.axis_index("c")
    base = wid * CH
    pltpu.sync_copy(p_hbm, p_v)
    pltpu.sync_copy(q_hbm, q_v)
    pltpu.sync_copy(s_hbm.at[pl.ds(base, CH)], s_v.at[pl.ds(0, CH)])
    pltpu.sync_copy(r_hbm.at[pl.ds(base, CH)], r_v.at[pl.ds(0, CH)])
    pltpu.sync_copy(le_hbm.at[pl.ds(base, CH)], le_v.at[pl.ds(0, CH)])

    iota = lax.iota(jnp.int32, 16)

    def logit_body(i, m16):
        sl = pl.ds(i * 16, 16)
        lg = (plsc.load_gather(p_v, [s_v[sl]])
              + plsc.load_gather(q_v, [r_v[sl]])
              + le_v[sl])
        lg = jnp.where(lg >= 0.0, lg, 0.2 * lg)
        lo_v[sl] = lg
        return jnp.maximum(m16, lg)

    m16 = lax.fori_loop(0, FULL, logit_body,
                        jnp.full((16,), -3e38, jnp.float32))

    tmask = iota < TAIL
    sl = pl.ds(FULL * 16, 16)
    s16 = jnp.where(tmask, s_v[sl], 0)
    r16 = jnp.where(tmask, r_v[sl], 0)
    lg = (plsc.load_gather(p_v, [s16], mask=tmask)
          + plsc.load_gather(q_v, [r16], mask=tmask)
          + jnp.where(tmask, le_v[sl], 0.0))
    lg = jnp.where(lg >= 0.0, lg, 0.2 * lg)
    lg = jnp.where(tmask, lg, -3e38)
    s_v[sl] = s16
    lo_v[sl] = lg
    m16 = jnp.maximum(m16, lg)

    ms = jnp.full((16,), jnp.max(m16))
    mx_v[...] = ms

    def zero_body(j, c):
        d_v[pl.ds(j * 16, 16)] = jnp.zeros((16,), jnp.float32)
        return c
    lax.fori_loop(0, NN // 16, zero_body, 0)

    def acc_body(i, c):
        sl = pl.ds(i * 16, 16)
        att = jnp.exp(lo_v[sl] - ms)
        plsc.addupdate_scatter(d_v, [s_v[sl]], att)
        return c
    lax.fori_loop(0, FULL + 1, acc_body, 0)

    pltpu.sync_copy(d_v, dpart_hbm.at[wid])
    pltpu.sync_copy(mx_v, mvec_hbm.at[wid])


def kernel(V, E, edges, W_f, W_a, b_a):
    V2 = V[0]
    E2 = E[0]

    le, p, q = pl.pallas_call(
        _k2_body,
        grid=(10,),
        in_specs=[pl.BlockSpec((16384, 128), lambda i: (i, 0)),
                  pl.BlockSpec((1024, 128), lambda i: (i, 0)),
                  pl.BlockSpec((128, 128), lambda i: (0, 0)),
                  pl.BlockSpec((1, 384), lambda i: (0, 0)),
                  pl.BlockSpec((1, 1), lambda i: (0, 0))],
        out_specs=[pl.BlockSpec((16384,), lambda i: (i,)),
                   pl.BlockSpec((1024,), lambda i: (i,)),
                   pl.BlockSpec((1024,), lambda i: (i,))],
        out_shape=[jax.ShapeDtypeStruct((NE,), jnp.float32),
                   jax.ShapeDtypeStruct((NN,), jnp.float32),
                   jax.ShapeDtypeStruct((NN,), jnp.float32)],
    )(E2, V2, W_f, W_a, b_a.reshape(1, 1))

    dpart, mvec = _k3(p, q, edges[0, :, 0], edges[0, :, 1], le)

    h = pl.pallas_call(
        _k4_body,
        out_shape=jax.ShapeDtypeStruct((NN, 128), jnp.float32),
    )(V2, W_f, dpart, mvec)
    return h.reshape(1, NN, DF)
